# Initial kernel scaffold; baseline (speedup 1.0000x reference)
import jax
import jax.numpy as jnp
from jax.experimental import pallas as pl


def kernel(x, up_index, down_index, boundary_index, coboundary_index, W_up1, b_up1, W_up2, b_up2, W_down1, b_down1, W_down2, b_down2, W_boundaries1, b_boundaries1, W_boundaries2, b_boundaries2, W_coboundaries1, b_coboundaries1, W_coboundaries2, b_coboundaries2, W_comb, b_comb):
    raise NotImplementedError("write your pallas kernel here")


# speedup vs baseline: 5.1099x; 5.1099x over previous
import functools

import jax
import jax.numpy as jnp
from jax import lax
from jax.experimental import pallas as pl
from jax.experimental.pallas import tpu as pltpu
from jax.experimental.pallas import tpu_sc as plsc

N = 10000
E = 320000
D = 128

NUM_CORES = 2
NUM_SUBCORES = 16
ADJ_PER_CORE = 2

CHUNK = 80
EDGES_PER_TILE = E // NUM_SUBCORES
NB = EDGES_PER_TILE // CHUNK
SUBNB = 25
NSTAGE = NB // SUBNB
ROWS_PER_TILE = 640
ROWS_LAST = N - 15 * ROWS_PER_TILE


def _sc_agg_body(x_hbm, src_hbm, dst_hbm, out_hbm,
                 acc, src_v, dst_v, rows_v, sem):
    c = lax.axis_index("c")
    s = lax.axis_index("s")
    row_base = pl.multiple_of(s * ROWS_PER_TILE, 8)

    for a_local in range(ADJ_PER_CORE):
        a = c * ADJ_PER_CORE + a_local

        @pl.when(s < NUM_SUBCORES - 1)
        def _():
            pltpu.sync_copy(x_hbm.at[pl.ds(row_base, ROWS_PER_TILE)],
                            acc.at[pl.ds(row_base, ROWS_PER_TILE)])

        @pl.when(s == NUM_SUBCORES - 1)
        def _():
            pltpu.sync_copy(x_hbm.at[pl.ds(N - ROWS_LAST, ROWS_LAST)],
                            acc.at[pl.ds(N - ROWS_LAST, ROWS_LAST)])

        plsc.subcore_barrier()

        def step(j, carry):
            pltpu.async_copy(x_hbm.at[src_v.at[j]], rows_v, sem).wait()
            pltpu.sync_copy(rows_v, acc.at[dst_v.at[j]], add=True)
            return carry

        for t in range(NSTAGE):
            pltpu.sync_copy(src_hbm.at[a, s, t], src_v)
            pltpu.sync_copy(dst_hbm.at[a, s, t], dst_v)
            lax.fori_loop(0, SUBNB, step, 0, unroll=False)
        plsc.subcore_barrier()

        @pl.when(s < NUM_SUBCORES - 1)
        def _():
            pltpu.sync_copy(acc.at[pl.ds(row_base, ROWS_PER_TILE)],
                            out_hbm.at[a, pl.ds(row_base, ROWS_PER_TILE)])

        @pl.when(s == NUM_SUBCORES - 1)
        def _():
            pltpu.sync_copy(acc.at[pl.ds(N - ROWS_LAST, ROWS_LAST)],
                            out_hbm.at[a, pl.ds(N - ROWS_LAST, ROWS_LAST)])

        plsc.subcore_barrier()


def _sc_aggregate(x, src_blocks, dst_blocks):
    mesh = plsc.VectorSubcoreMesh(core_axis_name="c", subcore_axis_name="s",
                                  num_cores=NUM_CORES,
                                  num_subcores=NUM_SUBCORES)
    f = pl.kernel(
        _sc_agg_body,
        out_type=jax.ShapeDtypeStruct((4, N, D), jnp.float32),
        mesh=mesh,
        scratch_types=[
            pltpu.VMEM_SHARED((N, D), jnp.float32),
            pltpu.VMEM((SUBNB, CHUNK), jnp.int32),
            pltpu.VMEM((SUBNB, CHUNK), jnp.int32),
            pltpu.VMEM((CHUNK, D), jnp.float32),
            pltpu.SemaphoreType.DMA,
        ],
    )
    return f(x, src_blocks, dst_blocks)


BN = 1000


def _tc_mlp_body(seg_ref, w1_ref, b1_ref, w2_ref, b2_ref, wc_ref, bc_ref,
                 out_ref):
    acc = jnp.zeros((BN, D), jnp.float32)
    for a in range(4):
        h = jnp.dot(seg_ref[a], w1_ref[a], preferred_element_type=jnp.float32)
        h = jnp.maximum(h + b1_ref[a], 0.0)
        h = jnp.dot(h, w2_ref[a], preferred_element_type=jnp.float32)
        h = h + b2_ref[a]
        acc = acc + jnp.dot(h, wc_ref[a], preferred_element_type=jnp.float32)
    out_ref[...] = jnp.maximum(acc + bc_ref[0], 0.0)


def _tc_mlp(seg, w1, b1, w2, b2, wc, bc):
    grid = (N // BN,)
    return pl.pallas_call(
        _tc_mlp_body,
        grid=grid,
        in_specs=[
            pl.BlockSpec((4, BN, D), lambda i: (0, i, 0)),
            pl.BlockSpec((4, D, D), lambda i: (0, 0, 0)),
            pl.BlockSpec((4, 1, D), lambda i: (0, 0, 0)),
            pl.BlockSpec((4, D, D), lambda i: (0, 0, 0)),
            pl.BlockSpec((4, 1, D), lambda i: (0, 0, 0)),
            pl.BlockSpec((4, D, D), lambda i: (0, 0, 0)),
            pl.BlockSpec((1, D), lambda i: (0, 0)),
        ],
        out_specs=pl.BlockSpec((BN, D), lambda i: (i, 0)),
        out_shape=jax.ShapeDtypeStruct((N, D), jnp.float32),
    )(seg, w1, b1, w2, b2, wc, bc)


def kernel(x, up_index, down_index, boundary_index, coboundary_index,
           W_up1, b_up1, W_up2, b_up2,
           W_down1, b_down1, W_down2, b_down2,
           W_boundaries1, b_boundaries1, W_boundaries2, b_boundaries2,
           W_coboundaries1, b_coboundaries1, W_coboundaries2, b_coboundaries2,
           W_comb, b_comb):
    idx = jnp.stack([up_index, down_index, boundary_index, coboundary_index])
    idx = idx.astype(jnp.int32)
    src_blocks = idx[:, 0, :].reshape(4, NUM_SUBCORES, NSTAGE, SUBNB, CHUNK)
    dst_blocks = idx[:, 1, :].reshape(4, NUM_SUBCORES, NSTAGE, SUBNB, CHUNK)

    seg = _sc_aggregate(x, src_blocks, dst_blocks)

    w1 = jnp.stack([W_up1, W_down1, W_boundaries1, W_coboundaries1])
    b1 = jnp.stack([b_up1, b_down1, b_boundaries1, b_coboundaries1])[:, None, :]
    w2 = jnp.stack([W_up2, W_down2, W_boundaries2, W_coboundaries2])
    b2 = jnp.stack([b_up2, b_down2, b_boundaries2, b_coboundaries2])[:, None, :]
    wc = W_comb.reshape(4, D, D)
    bc = b_comb[None, :]
    return _tc_mlp(seg, w1, b1, w2, b2, wc, bc)


# speedup vs baseline: 9.6369x; 1.8859x over previous
import functools

import jax
import jax.numpy as jnp
from jax import lax
from jax.experimental import pallas as pl
from jax.experimental.pallas import tpu as pltpu
from jax.experimental.pallas import tpu_sc as plsc

N = 10000
E = 320000
D = 128

NUM_CORES = 2
NUM_SUBCORES = 16
ADJ_PER_CORE = 2

CHUNK = 80
EDGES_PER_TILE = E // NUM_SUBCORES
NB = EDGES_PER_TILE // CHUNK
SUBNB = 50
NSTAGE = NB // SUBNB
NBUF = 3
NGROUP = (SUBNB + NBUF - 1) // NBUF
ROWS_PER_TILE = 640
ROWS_LAST = N - 15 * ROWS_PER_TILE


def _sc_agg_body(x_hbm, src_hbm, dst_hbm, out_hbm,
                 acc, src_v, dst_v, rows_v, sem0, sem1, sem2):
    c = lax.axis_index("c")
    s = lax.axis_index("s")
    row_base = pl.multiple_of(s * ROWS_PER_TILE, 8)
    sems = [sem0, sem1, sem2]

    def gather(j, b):
        return pltpu.async_copy(x_hbm.at[src_v.at[j]], rows_v.at[b], sems[b])

    for a_local in range(ADJ_PER_CORE):
        a = c * ADJ_PER_CORE + a_local

        @pl.when(s < NUM_SUBCORES - 1)
        def _():
            pltpu.sync_copy(x_hbm.at[pl.ds(row_base, ROWS_PER_TILE)],
                            acc.at[pl.ds(row_base, ROWS_PER_TILE)])

        @pl.when(s == NUM_SUBCORES - 1)
        def _():
            pltpu.sync_copy(x_hbm.at[pl.ds(N - ROWS_LAST, ROWS_LAST)],
                            acc.at[pl.ds(N - ROWS_LAST, ROWS_LAST)])

        plsc.subcore_barrier()

        for t in range(NSTAGE):
            pltpu.sync_copy(src_hbm.at[a, s, t], src_v)
            pltpu.sync_copy(dst_hbm.at[a, s, t], dst_v)
            gather(0, 0)
            gather(1, 1)

            def group(g, carry):
                for b in range(NBUF):
                    j = g * NBUF + b

                    @pl.when(j < SUBNB)
                    def _():
                        pltpu.make_async_copy(x_hbm.at[src_v.at[j]],
                                              rows_v.at[b], sems[b]).wait()

                        @pl.when(j + 2 < SUBNB)
                        def _():
                            gather(j + 2, (b + 2) % NBUF)

                        pltpu.sync_copy(rows_v.at[b], acc.at[dst_v.at[j]],
                                        add=True)
                return carry

            lax.fori_loop(0, NGROUP, group, 0, unroll=False)
        plsc.subcore_barrier()

        @pl.when(s < NUM_SUBCORES - 1)
        def _():
            pltpu.sync_copy(acc.at[pl.ds(row_base, ROWS_PER_TILE)],
                            out_hbm.at[a, pl.ds(row_base, ROWS_PER_TILE)])

        @pl.when(s == NUM_SUBCORES - 1)
        def _():
            pltpu.sync_copy(acc.at[pl.ds(N - ROWS_LAST, ROWS_LAST)],
                            out_hbm.at[a, pl.ds(N - ROWS_LAST, ROWS_LAST)])

        plsc.subcore_barrier()


def _sc_aggregate(x, src_blocks, dst_blocks):
    mesh = plsc.VectorSubcoreMesh(core_axis_name="c", subcore_axis_name="s",
                                  num_cores=NUM_CORES,
                                  num_subcores=NUM_SUBCORES)
    f = pl.kernel(
        _sc_agg_body,
        out_type=jax.ShapeDtypeStruct((4, N, D), jnp.float32),
        mesh=mesh,
        scratch_types=[
            pltpu.VMEM_SHARED((N, D), jnp.float32),
            pltpu.VMEM((SUBNB, CHUNK), jnp.int32),
            pltpu.VMEM((SUBNB, CHUNK), jnp.int32),
            pltpu.VMEM((NBUF, CHUNK, D), jnp.float32),
            pltpu.SemaphoreType.DMA,
            pltpu.SemaphoreType.DMA,
            pltpu.SemaphoreType.DMA,
        ],
    )
    return f(x, src_blocks, dst_blocks)


BN = 1000


def _tc_mlp_body(seg_ref, w1_ref, b1_ref, w2_ref, b2_ref, wc_ref, bc_ref,
                 out_ref):
    acc = jnp.zeros((BN, D), jnp.float32)
    for a in range(4):
        h = jnp.dot(seg_ref[a], w1_ref[a], preferred_element_type=jnp.float32)
        h = jnp.maximum(h + b1_ref[a], 0.0)
        h = jnp.dot(h, w2_ref[a], preferred_element_type=jnp.float32)
        h = h + b2_ref[a]
        acc = acc + jnp.dot(h, wc_ref[a], preferred_element_type=jnp.float32)
    out_ref[...] = jnp.maximum(acc + bc_ref[0], 0.0)


def _tc_mlp(seg, w1, b1, w2, b2, wc, bc):
    grid = (N // BN,)
    return pl.pallas_call(
        _tc_mlp_body,
        grid=grid,
        in_specs=[
            pl.BlockSpec((4, BN, D), lambda i: (0, i, 0)),
            pl.BlockSpec((4, D, D), lambda i: (0, 0, 0)),
            pl.BlockSpec((4, 1, D), lambda i: (0, 0, 0)),
            pl.BlockSpec((4, D, D), lambda i: (0, 0, 0)),
            pl.BlockSpec((4, 1, D), lambda i: (0, 0, 0)),
            pl.BlockSpec((4, D, D), lambda i: (0, 0, 0)),
            pl.BlockSpec((1, D), lambda i: (0, 0)),
        ],
        out_specs=pl.BlockSpec((BN, D), lambda i: (i, 0)),
        out_shape=jax.ShapeDtypeStruct((N, D), jnp.float32),
    )(seg, w1, b1, w2, b2, wc, bc)


def kernel(x, up_index, down_index, boundary_index, coboundary_index,
           W_up1, b_up1, W_up2, b_up2,
           W_down1, b_down1, W_down2, b_down2,
           W_boundaries1, b_boundaries1, W_boundaries2, b_boundaries2,
           W_coboundaries1, b_coboundaries1, W_coboundaries2, b_coboundaries2,
           W_comb, b_comb):
    idx = jnp.stack([up_index, down_index, boundary_index, coboundary_index])
    idx = idx.astype(jnp.int32)
    src_blocks = idx[:, 0, :].reshape(4, NUM_SUBCORES, NSTAGE, SUBNB, CHUNK)
    dst_blocks = idx[:, 1, :].reshape(4, NUM_SUBCORES, NSTAGE, SUBNB, CHUNK)

    seg = _sc_aggregate(x, src_blocks, dst_blocks)

    w1 = jnp.stack([W_up1, W_down1, W_boundaries1, W_coboundaries1])
    b1 = jnp.stack([b_up1, b_down1, b_boundaries1, b_coboundaries1])[:, None, :]
    w2 = jnp.stack([W_up2, W_down2, W_boundaries2, W_coboundaries2])
    b2 = jnp.stack([b_up2, b_down2, b_boundaries2, b_coboundaries2])[:, None, :]
    wc = W_comb.reshape(4, D, D)
    bc = b_comb[None, :]
    return _tc_mlp(seg, w1, b1, w2, b2, wc, bc)


# speedup vs baseline: 10.3657x; 1.0756x over previous
import functools

import jax
import jax.numpy as jnp
from jax import lax
from jax.experimental import pallas as pl
from jax.experimental.pallas import tpu as pltpu
from jax.experimental.pallas import tpu_sc as plsc

N = 10000
E = 320000
D = 128

NUM_CORES = 2
NUM_SUBCORES = 16
ADJ_PER_CORE = 2

CHUNK = 80
EDGES_PER_TILE = E // NUM_SUBCORES
NB = EDGES_PER_TILE // CHUNK
SUBNB = 50
NSTAGE = NB // SUBNB
NBUF = 3
NGROUP = (SUBNB + NBUF - 1) // NBUF
ROWS_PER_TILE = 640
ROWS_LAST = N - 15 * ROWS_PER_TILE


def _sc_agg_body(x_hbm, up_hbm, down_hbm, bnd_hbm, cob_hbm, out_hbm,
                 acc, src_v, dst_v, rows_v, sem0, sem1, sem2):
    c = lax.axis_index("c")
    s = lax.axis_index("s")
    row_base = pl.multiple_of(s * ROWS_PER_TILE, 8)
    sems = [sem0, sem1, sem2]

    def gather(j, b):
        return pltpu.async_copy(x_hbm.at[src_v.at[j]], rows_v.at[b], sems[b])

    for a_local in range(ADJ_PER_CORE):
        a = c * ADJ_PER_CORE + a_local
        idx_pair = [up_hbm, bnd_hbm] if a_local == 0 else [down_hbm, cob_hbm]

        @pl.when(s < NUM_SUBCORES - 1)
        def _():
            pltpu.sync_copy(x_hbm.at[pl.ds(row_base, ROWS_PER_TILE)],
                            acc.at[pl.ds(row_base, ROWS_PER_TILE)])

        @pl.when(s == NUM_SUBCORES - 1)
        def _():
            pltpu.sync_copy(x_hbm.at[pl.ds(N - ROWS_LAST, ROWS_LAST)],
                            acc.at[pl.ds(N - ROWS_LAST, ROWS_LAST)])

        plsc.subcore_barrier()

        for t in range(NSTAGE):
            for ci, idx_hbm in enumerate(idx_pair):
                @pl.when(c == ci)
                def _():
                    pltpu.sync_copy(idx_hbm.at[0, s, t], src_v)
                    pltpu.sync_copy(idx_hbm.at[1, s, t], dst_v)
            gather(0, 0)
            gather(1, 1)

            def group(g, carry):
                for b in range(NBUF):
                    j = g * NBUF + b

                    @pl.when(j < SUBNB)
                    def _():
                        pltpu.make_async_copy(x_hbm.at[src_v.at[j]],
                                              rows_v.at[b], sems[b]).wait()

                        @pl.when(j + 2 < SUBNB)
                        def _():
                            gather(j + 2, (b + 2) % NBUF)

                        pltpu.sync_copy(rows_v.at[b], acc.at[dst_v.at[j]],
                                        add=True)
                return carry

            lax.fori_loop(0, NGROUP, group, 0, unroll=False)
        plsc.subcore_barrier()

        @pl.when(s < NUM_SUBCORES - 1)
        def _():
            pltpu.sync_copy(acc.at[pl.ds(row_base, ROWS_PER_TILE)],
                            out_hbm.at[a, pl.ds(row_base, ROWS_PER_TILE)])

        @pl.when(s == NUM_SUBCORES - 1)
        def _():
            pltpu.sync_copy(acc.at[pl.ds(N - ROWS_LAST, ROWS_LAST)],
                            out_hbm.at[a, pl.ds(N - ROWS_LAST, ROWS_LAST)])

        plsc.subcore_barrier()


def _sc_aggregate(x, up_blocks, down_blocks, bnd_blocks, cob_blocks):
    mesh = plsc.VectorSubcoreMesh(core_axis_name="c", subcore_axis_name="s",
                                  num_cores=NUM_CORES,
                                  num_subcores=NUM_SUBCORES)
    f = pl.kernel(
        _sc_agg_body,
        out_type=jax.ShapeDtypeStruct((4, N, D), jnp.float32),
        mesh=mesh,
        scratch_types=[
            pltpu.VMEM_SHARED((N, D), jnp.float32),
            pltpu.VMEM((SUBNB, CHUNK), jnp.int32),
            pltpu.VMEM((SUBNB, CHUNK), jnp.int32),
            pltpu.VMEM((NBUF, CHUNK, D), jnp.float32),
            pltpu.SemaphoreType.DMA,
            pltpu.SemaphoreType.DMA,
            pltpu.SemaphoreType.DMA,
        ],
    )
    return f(x, up_blocks, down_blocks, bnd_blocks, cob_blocks)


BN = 1000


def _tc_mlp_body(seg_ref, w1_ref, b1_ref, w2_ref, b2_ref, wc_ref, bc_ref,
                 out_ref):
    acc = jnp.zeros((BN, D), jnp.float32)
    for a in range(4):
        h = jnp.dot(seg_ref[a], w1_ref[a], preferred_element_type=jnp.float32)
        h = jnp.maximum(h + b1_ref[a], 0.0)
        h = jnp.dot(h, w2_ref[a], preferred_element_type=jnp.float32)
        h = h + b2_ref[a]
        acc = acc + jnp.dot(h, wc_ref[a], preferred_element_type=jnp.float32)
    out_ref[...] = jnp.maximum(acc + bc_ref[0], 0.0)


def _tc_mlp(seg, w1, b1, w2, b2, wc, bc):
    grid = (N // BN,)
    return pl.pallas_call(
        _tc_mlp_body,
        grid=grid,
        in_specs=[
            pl.BlockSpec((4, BN, D), lambda i: (0, i, 0)),
            pl.BlockSpec((4, D, D), lambda i: (0, 0, 0)),
            pl.BlockSpec((4, 1, D), lambda i: (0, 0, 0)),
            pl.BlockSpec((4, D, D), lambda i: (0, 0, 0)),
            pl.BlockSpec((4, 1, D), lambda i: (0, 0, 0)),
            pl.BlockSpec((4, D, D), lambda i: (0, 0, 0)),
            pl.BlockSpec((1, D), lambda i: (0, 0)),
        ],
        out_specs=pl.BlockSpec((BN, D), lambda i: (i, 0)),
        out_shape=jax.ShapeDtypeStruct((N, D), jnp.float32),
    )(seg, w1, b1, w2, b2, wc, bc)


def kernel(x, up_index, down_index, boundary_index, coboundary_index,
           W_up1, b_up1, W_up2, b_up2,
           W_down1, b_down1, W_down2, b_down2,
           W_boundaries1, b_boundaries1, W_boundaries2, b_boundaries2,
           W_coboundaries1, b_coboundaries1, W_coboundaries2, b_coboundaries2,
           W_comb, b_comb):
    def blocks(idx):
        return idx.astype(jnp.int32).reshape(
            2, NUM_SUBCORES, NSTAGE, SUBNB, CHUNK)

    seg = _sc_aggregate(x, blocks(up_index), blocks(down_index),
                        blocks(boundary_index), blocks(coboundary_index))

    w1 = jnp.stack([W_up1, W_down1, W_boundaries1, W_coboundaries1])
    b1 = jnp.stack([b_up1, b_down1, b_boundaries1, b_coboundaries1])[:, None, :]
    w2 = jnp.stack([W_up2, W_down2, W_boundaries2, W_coboundaries2])
    b2 = jnp.stack([b_up2, b_down2, b_boundaries2, b_coboundaries2])[:, None, :]
    wc = W_comb.reshape(4, D, D)
    bc = b_comb[None, :]
    return _tc_mlp(seg, w1, b1, w2, b2, wc, bc)


# speedup vs baseline: 10.5288x; 1.0157x over previous
import functools

import jax
import jax.numpy as jnp
from jax import lax
from jax.experimental import pallas as pl
from jax.experimental.pallas import tpu as pltpu
from jax.experimental.pallas import tpu_sc as plsc

N = 10000
E = 320000
D = 128

NUM_CORES = 2
NUM_SUBCORES = 16
ADJ_PER_CORE = 2

CHUNK = 80
EDGES_PER_TILE = E // NUM_SUBCORES
NB = EDGES_PER_TILE // CHUNK
SUBNB = 50
NSTAGE = NB // SUBNB
NBUF = 3
NGROUP = (SUBNB + NBUF - 1) // NBUF
ROWS_PER_TILE = 640
ROWS_LAST = N - 15 * ROWS_PER_TILE


def _sc_agg_body(x_hbm, up_hbm, down_hbm, bnd_hbm, cob_hbm, out_hbm,
                 acc, src_v, dst_v, rows_v, sem0, sem1, sem2):
    c = lax.axis_index("c")
    s = lax.axis_index("s")
    row_base = pl.multiple_of(s * ROWS_PER_TILE, 8)
    sems = [sem0, sem1, sem2]

    def gather(j, b):
        return pltpu.async_copy(x_hbm.at[src_v.at[j]], rows_v.at[b], sems[b])

    for a_local in range(ADJ_PER_CORE):
        a = c * ADJ_PER_CORE + a_local
        idx_pair = [up_hbm, bnd_hbm] if a_local == 0 else [down_hbm, cob_hbm]

        @pl.when(s < NUM_SUBCORES - 1)
        def _():
            pltpu.sync_copy(x_hbm.at[pl.ds(row_base, ROWS_PER_TILE)],
                            acc.at[pl.ds(row_base, ROWS_PER_TILE)])

        @pl.when(s == NUM_SUBCORES - 1)
        def _():
            pltpu.sync_copy(x_hbm.at[pl.ds(N - ROWS_LAST, ROWS_LAST)],
                            acc.at[pl.ds(N - ROWS_LAST, ROWS_LAST)])

        plsc.subcore_barrier()

        for t in range(NSTAGE):
            for ci, idx_hbm in enumerate(idx_pair):
                @pl.when(c == ci)
                def _():
                    pltpu.sync_copy(idx_hbm.at[0, s, t], src_v)
                    pltpu.sync_copy(idx_hbm.at[1, s, t], dst_v)
            gather(0, 0)
            gather(1, 1)

            def group(g, carry):
                for b in range(NBUF):
                    j = g * NBUF + b

                    @pl.when(j < SUBNB)
                    def _():
                        pltpu.make_async_copy(x_hbm.at[src_v.at[j]],
                                              rows_v.at[b], sems[b]).wait()

                        @pl.when(j + 2 < SUBNB)
                        def _():
                            gather(j + 2, (b + 2) % NBUF)

                        pltpu.sync_copy(rows_v.at[b], acc.at[dst_v.at[j]],
                                        add=True)
                return carry

            lax.fori_loop(0, NGROUP, group, 0, unroll=False)
        plsc.subcore_barrier()

        @pl.when(s < NUM_SUBCORES - 1)
        def _():
            pltpu.sync_copy(acc.at[pl.ds(row_base, ROWS_PER_TILE)],
                            out_hbm.at[a, pl.ds(row_base, ROWS_PER_TILE)])

        @pl.when(s == NUM_SUBCORES - 1)
        def _():
            pltpu.sync_copy(acc.at[pl.ds(N - ROWS_LAST, ROWS_LAST)],
                            out_hbm.at[a, pl.ds(N - ROWS_LAST, ROWS_LAST)])

        plsc.subcore_barrier()


def _sc_aggregate(x, up_blocks, down_blocks, bnd_blocks, cob_blocks):
    mesh = plsc.VectorSubcoreMesh(core_axis_name="c", subcore_axis_name="s",
                                  num_cores=NUM_CORES,
                                  num_subcores=NUM_SUBCORES)
    f = pl.kernel(
        _sc_agg_body,
        out_type=jax.ShapeDtypeStruct((4, N, D), jnp.float32),
        mesh=mesh,
        scratch_types=[
            pltpu.VMEM_SHARED((N, D), jnp.float32),
            pltpu.VMEM((SUBNB, CHUNK), jnp.int32),
            pltpu.VMEM((SUBNB, CHUNK), jnp.int32),
            pltpu.VMEM((NBUF, CHUNK, D), jnp.float32),
            pltpu.SemaphoreType.DMA,
            pltpu.SemaphoreType.DMA,
            pltpu.SemaphoreType.DMA,
        ],
    )
    return f(x, up_blocks, down_blocks, bnd_blocks, cob_blocks)


BN = 2000


def _tc_mlp_body(seg_ref, w1_ref, b1_ref, w2_ref, b2_ref, wc_ref, bc_ref,
                 out_ref):
    acc = jnp.zeros((BN, D), jnp.float32)
    for a in range(4):
        h = jnp.dot(seg_ref[a], w1_ref[a], preferred_element_type=jnp.float32)
        h = jnp.maximum(h + b1_ref[a], 0.0)
        h = jnp.dot(h, w2_ref[a], preferred_element_type=jnp.float32)
        h = h + b2_ref[a]
        acc = acc + jnp.dot(h, wc_ref[a], preferred_element_type=jnp.float32)
    out_ref[...] = jnp.maximum(acc + bc_ref[0], 0.0)


def _tc_mlp(seg, w1, b1, w2, b2, wc, bc):
    grid = (N // BN,)
    return pl.pallas_call(
        _tc_mlp_body,
        grid=grid,
        in_specs=[
            pl.BlockSpec((4, BN, D), lambda i: (0, i, 0)),
            pl.BlockSpec((4, D, D), lambda i: (0, 0, 0)),
            pl.BlockSpec((4, 1, D), lambda i: (0, 0, 0)),
            pl.BlockSpec((4, D, D), lambda i: (0, 0, 0)),
            pl.BlockSpec((4, 1, D), lambda i: (0, 0, 0)),
            pl.BlockSpec((4, D, D), lambda i: (0, 0, 0)),
            pl.BlockSpec((1, D), lambda i: (0, 0)),
        ],
        out_specs=pl.BlockSpec((BN, D), lambda i: (i, 0)),
        out_shape=jax.ShapeDtypeStruct((N, D), jnp.float32),
    )(seg, w1, b1, w2, b2, wc, bc)


def kernel(x, up_index, down_index, boundary_index, coboundary_index,
           W_up1, b_up1, W_up2, b_up2,
           W_down1, b_down1, W_down2, b_down2,
           W_boundaries1, b_boundaries1, W_boundaries2, b_boundaries2,
           W_coboundaries1, b_coboundaries1, W_coboundaries2, b_coboundaries2,
           W_comb, b_comb):
    def blocks(idx):
        return idx.astype(jnp.int32).reshape(
            2, NUM_SUBCORES, NSTAGE, SUBNB, CHUNK)

    seg = _sc_aggregate(x, blocks(up_index), blocks(down_index),
                        blocks(boundary_index), blocks(coboundary_index))

    w1 = jnp.stack([W_up1, W_down1, W_boundaries1, W_coboundaries1])
    b1 = jnp.stack([b_up1, b_down1, b_boundaries1, b_coboundaries1])[:, None, :]
    w2 = jnp.stack([W_up2, W_down2, W_boundaries2, W_coboundaries2])
    b2 = jnp.stack([b_up2, b_down2, b_boundaries2, b_coboundaries2])[:, None, :]
    wc = W_comb.reshape(4, D, D)
    bc = b_comb[None, :]
    return _tc_mlp(seg, w1, b1, w2, b2, wc, bc)


# speedup vs baseline: 10.7727x; 1.0232x over previous
import jax
import jax.numpy as jnp
from jax import lax
from jax.experimental import pallas as pl
from jax.experimental.pallas import tpu as pltpu
from jax.experimental.pallas import tpu_sc as plsc

N = 10000
E = 320000
D = 128

NUM_CORES = 2
NUM_SUBCORES = 16

CHUNK = 80
EDGES_PER_TILE = E // NUM_SUBCORES
NB = EDGES_PER_TILE // CHUNK
SUBNB = 50
NSTAGE = NB // SUBNB
NBUF = 3
NGROUP = (SUBNB + NBUF - 1) // NBUF
ROWS_PER_TILE = 640
ROWS_LAST = N - 15 * ROWS_PER_TILE


def _sc_pair_body(x_hbm, idx0_hbm, idx1_hbm, out_hbm,
                  acc, src_v, dst_v, rows_v, sem0, sem1, sem2):
    c = lax.axis_index("c")
    s = lax.axis_index("s")
    row_base = pl.multiple_of(s * ROWS_PER_TILE, 8)
    sems = [sem0, sem1, sem2]

    def gather(j, b):
        return pltpu.async_copy(x_hbm.at[src_v.at[j]], rows_v.at[b], sems[b])

    @pl.when(s < NUM_SUBCORES - 1)
    def _():
        pltpu.sync_copy(x_hbm.at[pl.ds(row_base, ROWS_PER_TILE)],
                        acc.at[pl.ds(row_base, ROWS_PER_TILE)])

    @pl.when(s == NUM_SUBCORES - 1)
    def _():
        pltpu.sync_copy(x_hbm.at[pl.ds(N - ROWS_LAST, ROWS_LAST)],
                        acc.at[pl.ds(N - ROWS_LAST, ROWS_LAST)])

    plsc.subcore_barrier()

    for t in range(NSTAGE):
        for ci, idx_hbm in enumerate([idx0_hbm, idx1_hbm]):
            @pl.when(c == ci)
            def _():
                pltpu.sync_copy(idx_hbm.at[0, s, t], src_v)
                pltpu.sync_copy(idx_hbm.at[1, s, t], dst_v)
        gather(0, 0)
        gather(1, 1)

        def group(g, carry):
            for b in range(NBUF):
                j = g * NBUF + b

                @pl.when(j < SUBNB)
                def _():
                    pltpu.make_async_copy(x_hbm.at[src_v.at[j]],
                                          rows_v.at[b], sems[b]).wait()

                    @pl.when(j + 2 < SUBNB)
                    def _():
                        gather(j + 2, (b + 2) % NBUF)

                    pltpu.sync_copy(rows_v.at[b], acc.at[dst_v.at[j]],
                                    add=True)
            return carry

        lax.fori_loop(0, NGROUP, group, 0, unroll=False)
    plsc.subcore_barrier()

    @pl.when(s < NUM_SUBCORES - 1)
    def _():
        pltpu.sync_copy(acc.at[pl.ds(row_base, ROWS_PER_TILE)],
                        out_hbm.at[c, pl.ds(row_base, ROWS_PER_TILE)])

    @pl.when(s == NUM_SUBCORES - 1)
    def _():
        pltpu.sync_copy(acc.at[pl.ds(N - ROWS_LAST, ROWS_LAST)],
                        out_hbm.at[c, pl.ds(N - ROWS_LAST, ROWS_LAST)])


def _sc_round(x, idx0_blocks, idx1_blocks):
    mesh = plsc.VectorSubcoreMesh(core_axis_name="c", subcore_axis_name="s",
                                  num_cores=NUM_CORES,
                                  num_subcores=NUM_SUBCORES)
    f = pl.kernel(
        _sc_pair_body,
        out_type=jax.ShapeDtypeStruct((2, N, D), jnp.float32),
        mesh=mesh,
        scratch_types=[
            pltpu.VMEM_SHARED((N, D), jnp.float32),
            pltpu.VMEM((SUBNB, CHUNK), jnp.int32),
            pltpu.VMEM((SUBNB, CHUNK), jnp.int32),
            pltpu.VMEM((NBUF, CHUNK, D), jnp.float32),
            pltpu.SemaphoreType.DMA,
            pltpu.SemaphoreType.DMA,
            pltpu.SemaphoreType.DMA,
        ],
    )
    return f(x, idx0_blocks, idx1_blocks)


BN = 2000


def _tc_pair_partial_body(seg_ref, w1_ref, b1_ref, w2_ref, b2_ref, wc_ref,
                          out_ref):
    acc = jnp.zeros((BN, D), jnp.float32)
    for a in range(2):
        h = jnp.dot(seg_ref[a], w1_ref[a], preferred_element_type=jnp.float32)
        h = jnp.maximum(h + b1_ref[a], 0.0)
        h = jnp.dot(h, w2_ref[a], preferred_element_type=jnp.float32)
        h = h + b2_ref[a]
        acc = acc + jnp.dot(h, wc_ref[a], preferred_element_type=jnp.float32)
    out_ref[...] = acc


def _tc_pair_final_body(seg_ref, w1_ref, b1_ref, w2_ref, b2_ref, wc_ref,
                        bc_ref, part_ref, out_ref):
    acc = part_ref[...]
    for a in range(2):
        h = jnp.dot(seg_ref[a], w1_ref[a], preferred_element_type=jnp.float32)
        h = jnp.maximum(h + b1_ref[a], 0.0)
        h = jnp.dot(h, w2_ref[a], preferred_element_type=jnp.float32)
        h = h + b2_ref[a]
        acc = acc + jnp.dot(h, wc_ref[a], preferred_element_type=jnp.float32)
    out_ref[...] = jnp.maximum(acc + bc_ref[0], 0.0)


_SEG_SPEC = pl.BlockSpec((2, BN, D), lambda i: (0, i, 0))
_W_SPEC = pl.BlockSpec((2, D, D), lambda i: (0, 0, 0))
_B_SPEC = pl.BlockSpec((2, 1, D), lambda i: (0, 0, 0))
_ROW_SPEC = pl.BlockSpec((BN, D), lambda i: (i, 0))


def _tc_pair_partial(seg, w1, b1, w2, b2, wc):
    return pl.pallas_call(
        _tc_pair_partial_body,
        grid=(N // BN,),
        in_specs=[_SEG_SPEC, _W_SPEC, _B_SPEC, _W_SPEC, _B_SPEC, _W_SPEC],
        out_specs=_ROW_SPEC,
        out_shape=jax.ShapeDtypeStruct((N, D), jnp.float32),
    )(seg, w1, b1, w2, b2, wc)


def _tc_pair_final(seg, w1, b1, w2, b2, wc, bc, part):
    return pl.pallas_call(
        _tc_pair_final_body,
        grid=(N // BN,),
        in_specs=[_SEG_SPEC, _W_SPEC, _B_SPEC, _W_SPEC, _B_SPEC, _W_SPEC,
                  pl.BlockSpec((1, D), lambda i: (0, 0)), _ROW_SPEC],
        out_specs=_ROW_SPEC,
        out_shape=jax.ShapeDtypeStruct((N, D), jnp.float32),
    )(seg, w1, b1, w2, b2, wc, bc, part)


def kernel(x, up_index, down_index, boundary_index, coboundary_index,
           W_up1, b_up1, W_up2, b_up2,
           W_down1, b_down1, W_down2, b_down2,
           W_boundaries1, b_boundaries1, W_boundaries2, b_boundaries2,
           W_coboundaries1, b_coboundaries1, W_coboundaries2, b_coboundaries2,
           W_comb, b_comb):
    def blocks(idx):
        return idx.astype(jnp.int32).reshape(
            2, NUM_SUBCORES, NSTAGE, SUBNB, CHUNK)

    seg_r1 = _sc_round(x, blocks(up_index), blocks(boundary_index))
    seg_r2 = _sc_round(x, blocks(down_index), blocks(coboundary_index))

    wc = W_comb.reshape(4, D, D)

    w1_r1 = jnp.stack([W_up1, W_boundaries1])
    b1_r1 = jnp.stack([b_up1, b_boundaries1])[:, None, :]
    w2_r1 = jnp.stack([W_up2, W_boundaries2])
    b2_r1 = jnp.stack([b_up2, b_boundaries2])[:, None, :]
    wc_r1 = jnp.stack([wc[0], wc[2]])

    w1_r2 = jnp.stack([W_down1, W_coboundaries1])
    b1_r2 = jnp.stack([b_down1, b_coboundaries1])[:, None, :]
    w2_r2 = jnp.stack([W_down2, W_coboundaries2])
    b2_r2 = jnp.stack([b_down2, b_coboundaries2])[:, None, :]
    wc_r2 = jnp.stack([wc[1], wc[3]])

    part = _tc_pair_partial(seg_r1, w1_r1, b1_r1, w2_r1, b2_r1, wc_r1)
    return _tc_pair_final(seg_r2, w1_r2, b1_r2, w2_r2, b2_r2, wc_r2,
                          b_comb[None, :], part)
